# baseline (speedup 1.0000x reference)
import functools

import jax
import jax.numpy as jnp
from jax import lax
from jax.experimental import pallas as pl
from jax.experimental.pallas import tpu as pltpu
from jax.experimental.pallas import tpu_sc as plsc

_VOCAB_BLOCK = 2000
_NC, _NS = 2, 16
_NW = _NC * _NS
_CHUNK = 128


def _proj_body(e_ref, w_ref, o_ref):
    y = lax.dot_general(
        e_ref[...], w_ref[...],
        dimension_numbers=(((1,), (1,)), ((), ())),
        preferred_element_type=jnp.float32)
    yb = lax.bitcast_convert_type(
        y.astype(jnp.bfloat16).astype(jnp.float32), jnp.int32)
    h = yb.shape[1] // 2
    lo = lax.shift_right_logical(yb[:, :h], 16)
    hi = yb[:, h:] & jnp.int32(-65536)
    o_ref[...] = lo | hi


def _project_table(embed_table, W):
    V, D_in = embed_table.shape
    D_out = W.shape[0]
    return pl.pallas_call(
        _proj_body,
        grid=(V // _VOCAB_BLOCK,),
        in_specs=[
            pl.BlockSpec((_VOCAB_BLOCK, D_in), lambda i: (i, 0)),
            pl.BlockSpec((D_out, D_in), lambda i: (0, 0)),
        ],
        out_specs=pl.BlockSpec((_VOCAB_BLOCK, D_out // 2), lambda i: (i, 0)),
        out_shape=jax.ShapeDtypeStruct((V, D_out // 2), jnp.int32),
    )(embed_table, W)


_UNPACK_BLOCK = 2048


def _unpack_body(i_ref, o_ref):
    w = i_ref[...]
    lo = lax.bitcast_convert_type(lax.shift_left(w, 16), jnp.float32)
    hi = lax.bitcast_convert_type(w & jnp.int32(-65536), jnp.float32)
    d_out = o_ref.shape[1]
    h = w.shape[1]
    o_ref[...] = jnp.concatenate([lo, hi[:, :d_out - h]], axis=1)


def _unpack(gath, d_out):
    B, H = gath.shape
    return pl.pallas_call(
        _unpack_body,
        grid=(B // _UNPACK_BLOCK,),
        in_specs=[pl.BlockSpec((_UNPACK_BLOCK, H), lambda i: (i, 0))],
        out_specs=pl.BlockSpec((_UNPACK_BLOCK, d_out), lambda i: (i, 0)),
        out_shape=jax.ShapeDtypeStruct((B, d_out), jnp.float32),
    )(gath)


def _gather_rows(table, idx_flat):
    B = idx_flat.shape[0]
    D_pad = table.shape[1]
    b_per_w = B // _NW
    n_chunks = b_per_w // _CHUNK
    mesh = plsc.VectorSubcoreMesh(core_axis_name="c", subcore_axis_name="s")

    @functools.partial(
        pl.kernel,
        mesh=mesh,
        out_type=jax.ShapeDtypeStruct((B, D_pad), table.dtype),
        scratch_types=[
            pltpu.VMEM((b_per_w,), jnp.int32),
            pltpu.VMEM((_CHUNK, D_pad), table.dtype),
            pltpu.VMEM((_CHUNK, D_pad), table.dtype),
            pltpu.SemaphoreType.DMA,
            pltpu.SemaphoreType.DMA,
        ],
    )
    def k(table_hbm, idx_hbm, out_hbm, idx_v, rows0, rows1, sem0, sem1):
        wid = lax.axis_index("s") * _NC + lax.axis_index("c")
        base = wid * b_per_w
        bufs = ((rows0, sem0), (rows1, sem1))

        pltpu.sync_copy(idx_hbm.at[pl.ds(base, b_per_w)], idx_v)

        def idx_at(c):
            return idx_v.at[pl.ds(c * _CHUNK, _CHUNK)]

        pltpu.async_copy(table_hbm.at[idx_at(0)], rows0, sem0)

        def pair_body(p, carry):
            for b in (0, 1):
                cur, csem = bufs[b]
                nxt, nsem = bufs[1 - b]
                c = 2 * p + b

                @pl.when(c + 1 < n_chunks)
                def _():
                    pltpu.async_copy(table_hbm.at[idx_at(c + 1)], nxt, nsem)

                pltpu.make_async_copy(table_hbm.at[idx_at(c)], cur,
                                      csem).wait()
                pltpu.sync_copy(cur, out_hbm.at[pl.ds(base + c * _CHUNK,
                                                      _CHUNK)])
            return carry

        lax.fori_loop(0, n_chunks // 2, pair_body, 0)

    return k(table, idx_flat)


def kernel(x, embed_table, W):
    B, L = x.shape
    D_out = W.shape[0]
    n_words = -(-((D_out + 1) // 2) // 128) * 128
    d_elems = 2 * n_words
    W_pad = jnp.pad(W, ((0, d_elems - D_out), (0, 0)))
    packed = _project_table(embed_table, W_pad)
    gath = _gather_rows(packed, x.reshape(-1).astype(jnp.int32))
    lo = lax.bitcast_convert_type(lax.shift_left(gath, 16), jnp.float32)
    hi = lax.bitcast_convert_type(gath & jnp.int32(-65536), jnp.float32)
    out = jnp.concatenate([lo, hi[:, :D_out - n_words]], axis=1)
    return out.reshape(B, L, D_out)


# speedup vs baseline: 1.6499x; 1.6499x over previous
import functools

import jax
import jax.numpy as jnp
from jax import lax
from jax.experimental import pallas as pl
from jax.experimental.pallas import tpu as pltpu
from jax.experimental.pallas import tpu_sc as plsc

_VOCAB_BLOCK = 2000
_NC, _NS = 2, 16
_NW = _NC * _NS
_CHUNK = 128


def _proj_body(e_ref, w_ref, o_ref):
    o_ref[...] = lax.dot_general(
        e_ref[...], w_ref[...],
        dimension_numbers=(((1,), (1,)), ((), ())),
        preferred_element_type=jnp.float32)


def _project_table(embed_table, W):
    V, D_in = embed_table.shape
    D_out = W.shape[0]
    return pl.pallas_call(
        _proj_body,
        grid=(V // _VOCAB_BLOCK,),
        in_specs=[
            pl.BlockSpec((_VOCAB_BLOCK, D_in), lambda i: (i, 0)),
            pl.BlockSpec((D_out, D_in), lambda i: (0, 0)),
        ],
        out_specs=pl.BlockSpec((_VOCAB_BLOCK, D_out), lambda i: (i, 0)),
        out_shape=jax.ShapeDtypeStruct((V, D_out), jnp.float32),
    )(embed_table, W)


_COMPACT_BLOCK = 4096


def _compact_body(i_ref, o_ref):
    o_ref[...] = i_ref[:, :o_ref.shape[1]]


def _compact(out_pad, d_out):
    B, D_pad = out_pad.shape
    return pl.pallas_call(
        _compact_body,
        grid=(B // _COMPACT_BLOCK,),
        in_specs=[pl.BlockSpec((_COMPACT_BLOCK, D_pad), lambda i: (i, 0))],
        out_specs=pl.BlockSpec((_COMPACT_BLOCK, d_out), lambda i: (i, 0)),
        out_shape=jax.ShapeDtypeStruct((B, d_out), jnp.float32),
    )(out_pad)


def _gather_rows(table, idx_flat):
    B = idx_flat.shape[0]
    D_pad = table.shape[1]
    b_per_w = B // _NW
    n_chunks = b_per_w // _CHUNK
    mesh = plsc.VectorSubcoreMesh(core_axis_name="c", subcore_axis_name="s")

    @functools.partial(
        pl.kernel,
        mesh=mesh,
        out_type=jax.ShapeDtypeStruct((B, D_pad), jnp.float32),
        scratch_types=[
            pltpu.VMEM((b_per_w,), jnp.int32),
            pltpu.VMEM((_CHUNK, D_pad), jnp.float32),
            pltpu.VMEM((_CHUNK, D_pad), jnp.float32),
            pltpu.SemaphoreType.DMA,
            pltpu.SemaphoreType.DMA,
        ],
    )
    def k(table_hbm, idx_hbm, out_hbm, idx_v, rows0, rows1, sem0, sem1):
        wid = lax.axis_index("s") * _NC + lax.axis_index("c")
        base = wid * b_per_w
        bufs = ((rows0, sem0), (rows1, sem1))

        pltpu.sync_copy(idx_hbm.at[pl.ds(base, b_per_w)], idx_v)

        def idx_at(c):
            return idx_v.at[pl.ds(c * _CHUNK, _CHUNK)]

        pltpu.async_copy(table_hbm.at[idx_at(0)], rows0, sem0)

        def pair_body(p, carry):
            for b in (0, 1):
                cur, csem = bufs[b]
                nxt, nsem = bufs[1 - b]
                c = 2 * p + b

                @pl.when(c + 1 < n_chunks)
                def _():
                    pltpu.async_copy(table_hbm.at[idx_at(c + 1)], nxt, nsem)

                pltpu.make_async_copy(table_hbm.at[idx_at(c)], cur,
                                      csem).wait()
                pltpu.sync_copy(cur, out_hbm.at[pl.ds(base + c * _CHUNK,
                                                      _CHUNK)])
            return carry

        lax.fori_loop(0, n_chunks // 2, pair_body, 0)

    return k(table, idx_flat)


def kernel(x, embed_table, W):
    B, L = x.shape
    D_out = W.shape[0]
    d_pad = (-D_out) % 128
    W_pad = jnp.pad(W, ((0, d_pad), (0, 0)))
    proj = _project_table(embed_table, W_pad)
    out_pad = _gather_rows(proj, x.reshape(-1).astype(jnp.int32))
    return out_pad[:, :D_out].reshape(B, L, D_out)
